# baseline (speedup 1.0000x reference)
import functools

import jax
import jax.numpy as jnp
from jax import lax
from jax.experimental import pallas as pl
from jax.experimental.pallas import tpu as pltpu
from jax.experimental.pallas import tpu_sc as plsc

_B = 1024
_L = 1024
_E = 64
_BL = _B * _L

_NW = 32
_ROWS_PER_W = _BL // _NW
_C = 256
_NSUB = _C // 128
_NCH = _ROWS_PER_W // _C

_RSQRT_MAGIC = 0x5F3759DF


_GATHER_DNUMS = lax.GatherDimensionNumbers(
    offset_dims=(), collapsed_slice_dims=(0,), start_index_map=(0,))


def _hsum16(v):
    for m in (8, 4, 2, 1):
        idx = lax.iota(jnp.int32, 16) ^ m
        v = v + lax.gather(v, idx[:, None], dimension_numbers=_GATHER_DNUMS,
                           slice_sizes=(1,),
                           mode=lax.GatherScatterMode.PROMISE_IN_BOUNDS)
    return v


def _ln_rows(rows_v, pe_v, l_base, g_regs, b_regs):
    @pl.loop(0, _C)
    def _row(r):
        h = []
        for k in range(4):
            h.append(rows_v[r, pl.ds(16 * k, 16)] + pe_v[l_base + r, pl.ds(16 * k, 16)])
        s = (h[0] + h[1]) + (h[2] + h[3])
        mu_v = _hsum16(s) * (1.0 / 64.0)
        q = (h[0] * h[0] + h[1] * h[1]) + (h[2] * h[2] + h[3] * h[3])
        tv = _hsum16(q) * (1.0 / 64.0) - mu_v * mu_v + 1e-12
        yi = jnp.int32(_RSQRT_MAGIC) - (plsc.bitcast(tv, jnp.int32) >> 1)
        y = plsc.bitcast(yi, jnp.float32)
        xh = tv * 0.5
        for _ in range(3):
            y = y * (1.5 - xh * y * y)
        for k in range(4):
            rows_v[r, pl.ds(16 * k, 16)] = (h[k] - mu_v) * y * g_regs[k] + b_regs[k]


def _make_sc_call():
    mesh = plsc.VectorSubcoreMesh(core_axis_name="c", subcore_axis_name="s")

    @functools.partial(
        pl.kernel,
        out_type=jax.ShapeDtypeStruct((_BL, _E), jnp.float32),
        mesh=mesh,
        compiler_params=pltpu.CompilerParams(
            needs_layout_passes=False, use_tc_tiling_on_sc=False),
        scratch_types=[
            pltpu.VMEM((_L, _E), jnp.float32),
            pltpu.VMEM((_E,), jnp.float32),
            pltpu.VMEM((_E,), jnp.float32),
            pltpu.VMEM((_NSUB, 128), jnp.int32),
            pltpu.VMEM((_C, _E), jnp.float32),
            pltpu.SemaphoreType.DMA,
        ],
    )
    def sc_embed(x_hbm, table_hbm, pe_hbm, gamma_hbm, beta_hbm, out_hbm,
                 pe_v, g_v, b_v, idx_v, rows_v, sem):
        wid = lax.axis_index("s") * 2 + lax.axis_index("c")
        base = wid * _ROWS_PER_W

        pltpu.sync_copy(pe_hbm, pe_v)
        pltpu.sync_copy(gamma_hbm, g_v)
        pltpu.sync_copy(beta_hbm, b_v)
        g_regs = [g_v[pl.ds(16 * k, 16)] for k in range(4)]
        b_regs = [b_v[pl.ds(16 * k, 16)] for k in range(4)]

        @pl.loop(0, _NCH)
        def _chunk(c):
            cbase = base + c * _C
            pltpu.sync_copy(x_hbm.at[wid * _NCH + c], idx_v)
            for j in range(_NSUB):
                pltpu.async_copy(
                    table_hbm.at[idx_v.at[j]],
                    rows_v.at[pl.ds(j * 128, 128), :],
                    sem,
                )
            for j in range(_NSUB):
                pltpu.make_async_copy(
                    table_hbm.at[idx_v.at[j]],
                    rows_v.at[pl.ds(j * 128, 128), :],
                    sem,
                ).wait()
            l_base = (c * _C) % _L
            _ln_rows(rows_v, pe_v, l_base, g_regs, b_regs)
            pltpu.sync_copy(rows_v, out_hbm.at[pl.ds(cbase, _C), :])

    return sc_embed


_sc_embed = _make_sc_call()


@jax.jit
def kernel(x, table, pe, gamma, beta):
    x2 = x.reshape(_BL // (_NSUB * 128), _NSUB, 128)
    out = _sc_embed(x2, table, pe.reshape(_L, _E), gamma, beta)
    return out.reshape(_B, _L, _E)


# speedup vs baseline: 1.0946x; 1.0946x over previous
import functools

import jax
import jax.numpy as jnp
from jax import lax
from jax.experimental import pallas as pl
from jax.experimental.pallas import tpu as pltpu
from jax.experimental.pallas import tpu_sc as plsc

_B = 1024
_L = 1024
_E = 64
_BL = _B * _L

_NW = 32
_ROWS_PER_W = _BL // _NW
_C = 128
_NCH = _ROWS_PER_W // _C

_RSQRT_MAGIC = 0x5F3759DF

_GATHER_DNUMS = lax.GatherDimensionNumbers(
    offset_dims=(), collapsed_slice_dims=(0,), start_index_map=(0,))


def _hsum16(v):
    for m in (8, 4, 2, 1):
        idx = lax.iota(jnp.int32, 16) ^ m
        v = v + lax.gather(v, idx[:, None], dimension_numbers=_GATHER_DNUMS,
                           slice_sizes=(1,),
                           mode=lax.GatherScatterMode.PROMISE_IN_BOUNDS)
    return v


def _ln_rows(rows_v, pe_v, l_base, g_regs, b_regs):
    @pl.loop(0, _C, unroll=4)
    def _row(r):
        h = []
        for k in range(4):
            h.append(rows_v[r, pl.ds(16 * k, 16)] + pe_v[l_base + r, pl.ds(16 * k, 16)])
        s = (h[0] + h[1]) + (h[2] + h[3])
        mu_v = _hsum16(s) * (1.0 / 64.0)
        q = (h[0] * h[0] + h[1] * h[1]) + (h[2] * h[2] + h[3] * h[3])
        tv = _hsum16(q) * (1.0 / 64.0) - mu_v * mu_v + 1e-12
        yi = jnp.int32(_RSQRT_MAGIC) - (plsc.bitcast(tv, jnp.int32) >> 1)
        y = plsc.bitcast(yi, jnp.float32)
        xh = tv * 0.5
        for _ in range(3):
            y = y * (1.5 - xh * y * y)
        for k in range(4):
            rows_v[r, pl.ds(16 * k, 16)] = (h[k] - mu_v) * y * g_regs[k] + b_regs[k]


def _make_sc_call():
    mesh = plsc.VectorSubcoreMesh(core_axis_name="c", subcore_axis_name="s")

    @functools.partial(
        pl.kernel,
        out_type=jax.ShapeDtypeStruct((_BL, _E), jnp.float32),
        mesh=mesh,
        compiler_params=pltpu.CompilerParams(
            needs_layout_passes=False, use_tc_tiling_on_sc=False),
        scratch_types=[
            pltpu.VMEM((_L, _E), jnp.float32),
            pltpu.VMEM((_E,), jnp.float32),
            pltpu.VMEM((_E,), jnp.float32),
            pltpu.VMEM((_NCH, _C), jnp.int32),
            pltpu.VMEM((2, _C, _E), jnp.float32),
            pltpu.SemaphoreType.DMA((2,)),
            pltpu.SemaphoreType.DMA((2,)),
        ],
    )
    def sc_embed(x_hbm, table_hbm, pe_hbm, gamma_hbm, beta_hbm, out_hbm,
                 pe_v, g_v, b_v, idx_v, rows_v, gsem, osem):
        wid = lax.axis_index("s") * 2 + lax.axis_index("c")
        base = wid * _ROWS_PER_W

        pltpu.sync_copy(pe_hbm, pe_v)
        pltpu.sync_copy(gamma_hbm, g_v)
        pltpu.sync_copy(beta_hbm, b_v)
        pltpu.sync_copy(x_hbm.at[pl.ds(wid * _NCH, _NCH), :], idx_v)
        g_regs = [g_v[pl.ds(16 * k, 16)] for k in range(4)]
        b_regs = [b_v[pl.ds(16 * k, 16)] for k in range(4)]

        def fire_gather(c, p):
            pltpu.async_copy(table_hbm.at[idx_v.at[c]], rows_v.at[p], gsem.at[p])

        def wait_gather(p):
            pltpu.make_async_copy(
                table_hbm.at[idx_v.at[0]], rows_v.at[p], gsem.at[p]).wait()

        def fire_out(c, p):
            pltpu.async_copy(
                rows_v.at[p], out_hbm.at[pl.ds(base + c * _C, _C), :], osem.at[p])

        def wait_out(p):
            pltpu.make_async_copy(
                rows_v.at[p], out_hbm.at[pl.ds(0, _C), :], osem.at[p]).wait()

        fire_gather(0, 0)

        @pl.loop(0, _NCH, step=2)
        def _steady(c0):
            for p in (0, 1):
                c = c0 + p
                q = 1 - p

                @pl.when(c + 1 < _NCH)
                def _prefetch():
                    @pl.when(c >= 1)
                    def _drain_prev_out():
                        wait_out(q)
                    fire_gather(c + 1, q)

                wait_gather(p)
                l_base = (c * _C) & (_L - 1)
                _ln_rows(rows_v.at[p], pe_v, l_base, g_regs, b_regs)
                fire_out(c, p)

        wait_out(0)
        wait_out(1)

    return sc_embed


_sc_embed = _make_sc_call()


@jax.jit
def kernel(x, table, pe, gamma, beta):
    x2 = x.reshape(_BL // _C, _C)
    out = _sc_embed(x2, table, pe.reshape(_L, _E), gamma, beta)
    return out.reshape(_B, _L, _E)
